# baseline (speedup 1.0000x reference)
import functools

import jax
import jax.numpy as jnp
from jax import lax
from jax.experimental import pallas as pl
from jax.experimental.pallas import tpu as pltpu
from jax.experimental.pallas import tpu_sc as plsc

_EPS = 1e-12
_L = 16


def _allsum(x, perms):
    dnums = lax.GatherDimensionNumbers(
        offset_dims=(), collapsed_slice_dims=(0,), start_index_map=(0,))
    for idx in perms:
        x = x + lax.gather(x, idx[:, None], dnums, slice_sizes=(1,),
                           mode=lax.GatherScatterMode.PROMISE_IN_BOUNDS)
    return x


def _rsqrt_newton(v):
    i = lax.bitcast_convert_type(v, jnp.int32)
    y = lax.bitcast_convert_type(
        jnp.int32(0x5F3759DF) - lax.shift_right_arithmetic(i, 1), jnp.float32)
    for _ in range(3):
        y = y * (1.5 - 0.5 * v * y * y)
    return y


def _make_fused(total_rows, hidden, seq, num_workers=32, chunk=128):
    rows_pw = total_rows // num_workers
    n_chunks = rows_pw // chunk
    idx_rows_pw = rows_pw // chunk
    pos_tiles = seq // rows_pw
    n_c = hidden // _L

    mesh = plsc.VectorSubcoreMesh(core_axis_name="c", subcore_axis_name="s")

    @functools.partial(
        pl.kernel,
        mesh=mesh,
        out_type=jax.ShapeDtypeStruct((total_rows, hidden), jnp.float32),
        scratch_types=[
            pltpu.VMEM((idx_rows_pw, chunk), jnp.int32),
            pltpu.VMEM((idx_rows_pw, chunk), jnp.int32),
            pltpu.VMEM((rows_pw, hidden), jnp.float32),
            pltpu.VMEM((rows_pw, hidden), jnp.float32),
            pltpu.VMEM((rows_pw, hidden), jnp.float32),
            pltpu.VMEM((2, hidden), jnp.float32),
            pltpu.SemaphoreType.DMA,
        ],
    )
    def fused(ids_hbm, sids_hbm, word_hbm, pos_hbm, seg_hbm, gb_hbm, out_hbm,
              idx_v, sid_v, words_v, pos_v, segr_v, gb_v, sem):
        wid = lax.axis_index("s") * 2 + lax.axis_index("c")
        base = wid * rows_pw
        pltpu.sync_copy(ids_hbm.at[pl.ds(wid * idx_rows_pw, idx_rows_pw)],
                        idx_v)
        pltpu.sync_copy(sids_hbm.at[pl.ds(wid * idx_rows_pw, idx_rows_pw)],
                        sid_v)
        copies = []
        for j in range(n_chunks):
            copies.append(pltpu.async_copy(
                word_hbm.at[idx_v.at[j]],
                words_v.at[pl.ds(j * chunk, chunk)], sem))
        for j in range(n_chunks):
            copies.append(pltpu.async_copy(
                seg_hbm.at[sid_v.at[j]],
                segr_v.at[pl.ds(j * chunk, chunk)], sem))
        pos_base = lax.rem(wid, pos_tiles) * rows_pw
        copies.append(pltpu.async_copy(
            pos_hbm.at[pl.ds(pos_base, rows_pw)], pos_v, sem))
        copies.append(pltpu.async_copy(gb_hbm, gb_v, sem))
        for cp in copies:
            cp.wait()

        gs = [gb_v[0, pl.ds(c * _L, _L)] for c in range(n_c)]
        bs = [gb_v[1, pl.ds(c * _L, _L)] for c in range(n_c)]
        inv_h = jnp.float32(1.0 / hidden)
        lanes = lax.iota(jnp.int32, _L)
        perms = [lax.bitwise_xor(lanes, jnp.int32(k)) for k in (8, 4, 2, 1)]

        def one_token(t):
            xs = []
            acc_s = jnp.zeros((_L,), jnp.float32)
            acc_q = jnp.zeros((_L,), jnp.float32)
            for c in range(n_c):
                sl = pl.ds(c * _L, _L)
                x = words_v[t, sl] + pos_v[t, sl] + segr_v[t, sl]
                xs.append(x)
                acc_s = acc_s + x
                acc_q = acc_q + x * x
            mv = _allsum(acc_s, perms) * inv_h
            var = _allsum(acc_q, perms) * inv_h - mv * mv
            inv = _rsqrt_newton(var + _EPS)
            for c in range(n_c):
                sl = pl.ds(c * _L, _L)
                words_v[t, sl] = (xs[c] - mv) * inv * gs[c] + bs[c]

        def body(i, carry):
            one_token(i * 2)
            one_token(i * 2 + 1)
            return carry

        lax.fori_loop(0, rows_pw // 2, body, jnp.int32(0))
        pltpu.sync_copy(words_v, out_hbm.at[pl.ds(base, rows_pw)])

    return fused


def kernel(input_ids, segment_ids, word_emb, pos_emb, seg_emb, gamma, beta):
    batch, seq = input_ids.shape
    hidden = word_emb.shape[1]
    total = batch * seq
    ids_flat = input_ids.reshape(total // 128, 128).astype(jnp.int32)
    sids_flat = segment_ids.reshape(total // 128, 128).astype(jnp.int32)
    gb = jnp.stack([gamma, beta]).astype(jnp.float32)
    out = _make_fused(total, hidden, seq)(
        ids_flat, sids_flat, word_emb, pos_emb, seg_emb, gb)
    return out.reshape(batch, seq, hidden)
